# Initial kernel scaffold; baseline (speedup 1.0000x reference)
import functools

import jax
import jax.numpy as jnp
from jax import lax
from jax.experimental import pallas as pl
from jax.experimental.pallas import tpu as pltpu
from jax.experimental.pallas import tpu_sc as plsc

B, S = 16384, 200
V, D = 25006, 100
VPAD = 25088
VB = VPAD // 8
NC, NS = 2, 16
NW = NC * NS
RPW = B // NW
ROW_W = S
XW = RPW * ROW_W


def _proj_kernel(t_ref, w_ref, b_ref, o_ref):
    t = t_ref[...]
    w = w_ref[...]
    s = jnp.sum(t * w, axis=1, keepdims=True)
    o_ref[...] = (s + b_ref[0, 0]) * (1.0 / S)


def _project_table(emb_table, W, b):
    return pl.pallas_call(
        _proj_kernel,
        grid=(VPAD // VB,),
        in_specs=[
            pl.BlockSpec((VB, D), lambda i: (i, 0)),
            pl.BlockSpec((1, D), lambda i: (0, 0)),
            pl.BlockSpec((1, 1), lambda i: (0, 0)),
        ],
        out_specs=pl.BlockSpec((VB, 1), lambda i: (i, 0)),
        out_shape=jax.ShapeDtypeStruct((VPAD, 1), jnp.float32),
    )(emb_table, W, b.reshape(1, 1))


def _sc_body(p_hbm, x_hbm, out_hbm, p_v, x_v, out_v):
    wid = lax.axis_index("s") * NC + lax.axis_index("c")
    base = wid * XW

    pltpu.sync_copy(p_hbm, p_v)
    pltpu.sync_copy(x_hbm.at[pl.ds(base, XW)], x_v.at[pl.ds(0, XW)])
    x_v[pl.ds(XW, 16)] = jnp.zeros((16,), jnp.int32)

    lane = lax.iota(jnp.int32, 16)
    head = lane < (S - 12 * 16)

    def row_body(r, carry):
        off = r * ROW_W
        acc = jnp.zeros((16,), jnp.float32)
        for j in range(12):
            idx = x_v[pl.ds(off + j * 16, 16)]
            acc = acc + plsc.load_gather(p_v, [idx])
        idx_t = x_v[pl.ds(off + 192, 16)]
        g = plsc.load_gather(p_v, [idx_t])
        acc = acc + jnp.where(head, g, 0.0)
        out_v[r] = jnp.sum(acc)
        return carry

    lax.fori_loop(0, RPW, row_body, 0)
    pltpu.sync_copy(out_v, out_hbm.at[pl.ds(wid * RPW, RPW)])


_sc_gather_sum = functools.partial(
    pl.kernel,
    mesh=plsc.VectorSubcoreMesh(core_axis_name="c", subcore_axis_name="s"),
    out_type=jax.ShapeDtypeStruct((B,), jnp.float32),
    scratch_types=[
        pltpu.VMEM((VPAD,), jnp.float32),
        pltpu.VMEM((XW + 16,), jnp.int32),
        pltpu.VMEM((RPW,), jnp.float32),
    ],
)(_sc_body)


def kernel(x, emb_table, W, b):
    p = _project_table(emb_table, W, b).reshape(VPAD)
    out = _sc_gather_sum(p, x.reshape(B * S))
    return out.reshape(B, 1, 1, 1)


# speedup vs baseline: 132.2527x; 132.2527x over previous
import functools

import jax
import jax.numpy as jnp
from jax import lax
from jax.experimental import pallas as pl
from jax.experimental.pallas import tpu as pltpu
from jax.experimental.pallas import tpu_sc as plsc

B, S = 16384, 200
V, D = 25006, 100
VPAD = 25088
VB = VPAD // 8
NC, NS = 2, 16
NW = NC * NS
RPW = B // NW
ROW_W = S
XW = RPW * ROW_W


def _proj_kernel(t_ref, w_ref, b_ref, o_ref):
    t = t_ref[...]
    w = w_ref[...]
    s = jnp.sum(t * w, axis=1, keepdims=True)
    o_ref[...] = (s + b_ref[0, 0]) * (1.0 / S)


def _project_table(emb_table, W, b):
    return pl.pallas_call(
        _proj_kernel,
        grid=(VPAD // VB,),
        in_specs=[
            pl.BlockSpec((VB, D), lambda i: (i, 0)),
            pl.BlockSpec((1, D), lambda i: (0, 0)),
            pl.BlockSpec((1, 1), lambda i: (0, 0)),
        ],
        out_specs=pl.BlockSpec((VB, 1), lambda i: (i, 0)),
        out_shape=jax.ShapeDtypeStruct((VPAD, 1), jnp.float32),
    )(emb_table, W, b.reshape(1, 1))


def _sc_body(p_hbm, x_hbm, out_hbm, p_v, x_v, out_v):
    wid = lax.axis_index("s") * NC + lax.axis_index("c")
    base = wid * XW

    pltpu.sync_copy(p_hbm, p_v)
    pltpu.sync_copy(x_hbm.at[pl.ds(base, XW)], x_v.at[pl.ds(0, XW)])
    x_v[pl.ds(XW, 16)] = jnp.zeros((16,), jnp.int32)

    lane = lax.iota(jnp.int32, 16)
    head = lane < (S - 12 * 16)

    def row_body(r, carry):
        off = r * ROW_W
        acc = jnp.zeros((16,), jnp.float32)
        for j in range(12):
            idx = x_v[pl.ds(off + j * 16, 16)]
            acc = acc + plsc.load_gather(p_v, [idx])
        idx_t = x_v[pl.ds(off + 192, 16)]
        g = plsc.load_gather(p_v, [idx_t])
        acc = acc + jnp.where(head, g, 0.0)
        csum = plsc.cumsum(acc)
        plsc.store_scatter(out_v, [jnp.full((16,), r, jnp.int32)], csum,
                           mask=lane == 15)
        return carry

    lax.fori_loop(0, RPW, row_body, 0)
    pltpu.sync_copy(out_v, out_hbm.at[pl.ds(wid * RPW, RPW)])


_sc_gather_sum = functools.partial(
    pl.kernel,
    mesh=plsc.VectorSubcoreMesh(core_axis_name="c", subcore_axis_name="s"),
    out_type=jax.ShapeDtypeStruct((B,), jnp.float32),
    compiler_params=pltpu.CompilerParams(needs_layout_passes=False),
    scratch_types=[
        pltpu.VMEM((VPAD,), jnp.float32),
        pltpu.VMEM((XW + 16,), jnp.int32),
        pltpu.VMEM((RPW,), jnp.float32),
    ],
)(_sc_body)


def kernel(x, emb_table, W, b):
    p = _project_table(emb_table, W, b).reshape(VPAD)
    out = _sc_gather_sum(p, x.reshape(B * S))
    return out.reshape(B, 1, 1, 1)


# speedup vs baseline: 158.2043x; 1.1962x over previous
import functools

import jax
import jax.numpy as jnp
from jax import lax
from jax.experimental import pallas as pl
from jax.experimental.pallas import tpu as pltpu
from jax.experimental.pallas import tpu_sc as plsc

B, S = 16384, 200
V, D = 25006, 100
VPAD = 25088
VB = VPAD // 8
NC, NS = 2, 16
NW = NC * NS
RPW = B // NW
ROW_W = S
XW = RPW * ROW_W


def _proj_kernel(t_ref, w_ref, b_ref, o_ref):
    t = t_ref[...]
    w = w_ref[...]
    s = jnp.sum(t * w, axis=1, keepdims=True)
    o_ref[...] = (s + b_ref[0, 0]) * (1.0 / S)


def _project_table(emb_table, W, b):
    return pl.pallas_call(
        _proj_kernel,
        grid=(VPAD // VB,),
        in_specs=[
            pl.BlockSpec((VB, D), lambda i: (i, 0)),
            pl.BlockSpec((1, D), lambda i: (0, 0)),
            pl.BlockSpec((1, 1), lambda i: (0, 0)),
        ],
        out_specs=pl.BlockSpec((VB, 1), lambda i: (i, 0)),
        out_shape=jax.ShapeDtypeStruct((VPAD, 1), jnp.float32),
    )(emb_table, W, b.reshape(1, 1))


CH = 128
NCH = RPW // CH


def _sc_body(p_hbm, x_hbm, out_hbm, p_v, x_v, out_v, sem_p, sem_a, sem_b):
    wid = lax.axis_index("s") * NC + lax.axis_index("c")
    row0 = wid * RPW

    cp_p = pltpu.async_copy(p_hbm, p_v, sem_p)
    sems = (sem_a, sem_b)
    cps = [None, None]
    cps[0] = pltpu.async_copy(x_hbm.at[pl.ds(row0, CH)], x_v.at[0], sem_a)

    lane = lax.iota(jnp.int32, 16)
    tail = lane >= 8

    for c in range(NCH):
        buf = c & 1
        if c + 1 < NCH:
            nbuf = (c + 1) & 1
            cps[nbuf] = pltpu.async_copy(
                x_hbm.at[pl.ds(row0 + (c + 1) * CH, CH)], x_v.at[nbuf],
                sems[nbuf])
        cps[buf].wait()
        if c == 0:
            cp_p.wait()

        def row_body(r, carry, _buf=buf, _c=c):
            acc = jnp.zeros((16,), jnp.float32)
            for j in range(12):
                idx = x_v[_buf, r, pl.ds(j * 16, 16)]
                acc = acc + plsc.load_gather(p_v, [idx])
            idx_t = x_v[_buf, r, pl.ds(184, 16)]
            g = plsc.load_gather(p_v, [idx_t])
            acc = acc + jnp.where(tail, g, 0.0)
            csum = plsc.cumsum(acc)
            plsc.store_scatter(out_v,
                               [jnp.full((16,), _c * CH + r, jnp.int32)],
                               csum, mask=lane == 15)
            return carry

        lax.fori_loop(0, CH, row_body, 0, unroll=2)

    pltpu.sync_copy(out_v, out_hbm.at[pl.ds(wid * RPW, RPW)])


_sc_gather_sum = functools.partial(
    pl.kernel,
    mesh=plsc.VectorSubcoreMesh(core_axis_name="c", subcore_axis_name="s"),
    out_type=jax.ShapeDtypeStruct((B,), jnp.float32),
    compiler_params=pltpu.CompilerParams(needs_layout_passes=False),
    scratch_types=[
        pltpu.VMEM((VPAD,), jnp.float32),
        pltpu.VMEM((2, CH, S), jnp.int32),
        pltpu.VMEM((RPW,), jnp.float32),
        pltpu.SemaphoreType.DMA,
        pltpu.SemaphoreType.DMA,
        pltpu.SemaphoreType.DMA,
    ],
)(_sc_body)


def kernel(x, emb_table, W, b):
    p = _project_table(emb_table, W, b).reshape(VPAD)
    out = _sc_gather_sum(p, x)
    return out.reshape(B, 1, 1, 1)


# speedup vs baseline: 269.1199x; 1.7011x over previous
import functools

import jax
import jax.numpy as jnp
from jax import lax
from jax.experimental import pallas as pl
from jax.experimental.pallas import tpu as pltpu
from jax.experimental.pallas import tpu_sc as plsc

B, S = 16384, 200
V, D = 25006, 100
VPAD = 25600
VB = 1024
NC, NS = 2, 16
NW = NC * NS
CPW = B // NW
CH = 128
NCH = CPW // CH
NG = CH // 16


def _proj_kernel(t_ref, w_ref, b_ref, o_ref):
    t = t_ref[...]
    w = w_ref[...]
    s = jnp.sum(t * w, axis=0)
    o_ref[...] = (s + b_ref[0, 0]) * (1.0 / S)


def _project_table(table_t, w_col, b):
    return pl.pallas_call(
        _proj_kernel,
        grid=(VPAD // VB,),
        in_specs=[
            pl.BlockSpec((D, VB), lambda i: (0, i)),
            pl.BlockSpec((D, 1), lambda i: (0, 0)),
            pl.BlockSpec((1, 1), lambda i: (0, 0)),
        ],
        out_specs=pl.BlockSpec((VB,), lambda i: (i,)),
        out_shape=jax.ShapeDtypeStruct((VPAD,), jnp.float32),
    )(table_t, w_col, b.reshape(1, 1))


def _sc_body(p_hbm, xt_hbm, out_hbm, p_v, x_v, out_v, sem_p, sem_a, sem_b):
    wid = lax.axis_index("s") * NC + lax.axis_index("c")
    col0 = wid * CPW

    cp_p = pltpu.async_copy(p_hbm, p_v, sem_p)
    sems = (sem_a, sem_b)
    cps = [None, None]
    cps[0] = pltpu.async_copy(xt_hbm.at[:, pl.ds(col0, CH)], x_v.at[0], sem_a)

    for c in range(NCH):
        buf = c & 1
        if c + 1 < NCH:
            nbuf = (c + 1) & 1
            cps[nbuf] = pltpu.async_copy(
                xt_hbm.at[:, pl.ds(col0 + (c + 1) * CH, CH)], x_v.at[nbuf],
                sems[nbuf])
        cps[buf].wait()
        if c == 0:
            cp_p.wait()

        for g in range(NG):
            def s_body(s, acc, _buf=buf, _g=g):
                idx = x_v[_buf, s, pl.ds(_g * 16, 16)]
                return acc + plsc.load_gather(p_v, [idx])

            acc = lax.fori_loop(0, S, s_body, jnp.zeros((16,), jnp.float32),
                                unroll=8)
            out_v[pl.ds(c * CH + g * 16, 16)] = acc

    pltpu.sync_copy(out_v, out_hbm.at[pl.ds(col0, CPW)])


_sc_gather_sum = functools.partial(
    pl.kernel,
    mesh=plsc.VectorSubcoreMesh(core_axis_name="c", subcore_axis_name="s"),
    out_type=jax.ShapeDtypeStruct((B,), jnp.float32),
    compiler_params=pltpu.CompilerParams(needs_layout_passes=False),
    scratch_types=[
        pltpu.VMEM((VPAD,), jnp.float32),
        pltpu.VMEM((2, S, CH), jnp.int32),
        pltpu.VMEM((CPW,), jnp.float32),
        pltpu.SemaphoreType.DMA,
        pltpu.SemaphoreType.DMA,
        pltpu.SemaphoreType.DMA,
    ],
)(_sc_body)


def kernel(x, emb_table, W, b):
    p = _project_table(emb_table.T, W.T, b)
    out = _sc_gather_sum(p, x.T)
    return out.reshape(B, 1, 1, 1)


# speedup vs baseline: 335.2333x; 1.2457x over previous
import functools

import jax
import jax.numpy as jnp
from jax import lax
from jax.experimental import pallas as pl
from jax.experimental.pallas import tpu as pltpu
from jax.experimental.pallas import tpu_sc as plsc

B, S = 16384, 200
V, D = 25006, 100
VPAD = 25600
VB = 1024
NC, NS = 2, 16
NW = NC * NS
CPW = B // NW
CH = 128
NCH = CPW // CH
NG = CH // 16


def _proj_kernel(t_ref, w_ref, b_ref, o_ref):
    t = t_ref[...]
    w = w_ref[...]
    s = jnp.dot(w, t, preferred_element_type=jnp.float32)
    o_ref[...] = (s.reshape(VPAD) + b_ref[0, 0]) * (1.0 / S)


def _project_table(table_t, w_row, b):
    return pl.pallas_call(
        _proj_kernel,
        grid=(1,),
        in_specs=[
            pl.BlockSpec((D, VPAD), lambda i: (0, 0)),
            pl.BlockSpec((1, D), lambda i: (0, 0)),
            pl.BlockSpec((1, 1), lambda i: (0, 0)),
        ],
        out_specs=pl.BlockSpec((VPAD,), lambda i: (0,)),
        out_shape=jax.ShapeDtypeStruct((VPAD,), jnp.float32),
    )(table_t, w_row, b.reshape(1, 1))


def _sc_body(p_hbm, xt_hbm, out_hbm, p_v, x_v, out_v, sem_p, sem_a, sem_b):
    wid = lax.axis_index("s") * NC + lax.axis_index("c")
    col0 = wid * CPW

    cp_p = pltpu.async_copy(p_hbm, p_v, sem_p)
    sems = (sem_a, sem_b)
    cps = [None, None]
    cps[0] = pltpu.async_copy(xt_hbm.at[:, pl.ds(col0, CH)], x_v.at[0], sem_a)

    for c in range(NCH):
        buf = c & 1
        if c + 1 < NCH:
            nbuf = (c + 1) & 1
            cps[nbuf] = pltpu.async_copy(
                xt_hbm.at[:, pl.ds(col0 + (c + 1) * CH, CH)], x_v.at[nbuf],
                sems[nbuf])
        cps[buf].wait()
        if c == 0:
            cp_p.wait()

        for g in range(NG):
            def s_body(s, acc, _buf=buf, _g=g):
                idx = x_v[_buf, s, pl.ds(_g * 16, 16)]
                return acc + plsc.load_gather(p_v, [idx])

            acc = lax.fori_loop(0, S, s_body, jnp.zeros((16,), jnp.float32),
                                unroll=20)
            out_v[pl.ds(c * CH + g * 16, 16)] = acc

    pltpu.sync_copy(out_v, out_hbm.at[pl.ds(col0, CPW)])


_sc_gather_sum = functools.partial(
    pl.kernel,
    mesh=plsc.VectorSubcoreMesh(core_axis_name="c", subcore_axis_name="s"),
    out_type=jax.ShapeDtypeStruct((B,), jnp.float32),
    compiler_params=pltpu.CompilerParams(needs_layout_passes=False),
    scratch_types=[
        pltpu.VMEM((VPAD,), jnp.float32),
        pltpu.VMEM((2, S, CH), jnp.int32),
        pltpu.VMEM((CPW,), jnp.float32),
        pltpu.SemaphoreType.DMA,
        pltpu.SemaphoreType.DMA,
        pltpu.SemaphoreType.DMA,
    ],
)(_sc_body)


def kernel(x, emb_table, W, b):
    p = _project_table(emb_table.T, W, b)
    out = _sc_gather_sum(p, x.T)
    return out.reshape(B, 1, 1, 1)


# speedup vs baseline: 363.2170x; 1.0835x over previous
import functools

import jax
import jax.numpy as jnp
from jax import lax
from jax.experimental import pallas as pl
from jax.experimental.pallas import tpu as pltpu
from jax.experimental.pallas import tpu_sc as plsc

B, S = 16384, 200
V, D = 25006, 100
VPAD = 25600
NC, NS = 2, 16
NW = NC * NS
CPW = B // NW
CH = 128
NCH = CPW // CH
NG = CH // 16


PG = 1
PB = VPAD // PG


def _proj_kernel(t_ref, w_ref, b_ref, o_ref):
    t = t_ref[...]
    w = w_ref[...]
    s = jnp.dot(w, t, preferred_element_type=jnp.float32)
    o_ref[...] = (s.reshape(PB) + b_ref[0, 0]) * (1.0 / S)


def _project_table(table_t, w_row, b):
    return pl.pallas_call(
        _proj_kernel,
        grid=(PG,),
        in_specs=[
            pl.BlockSpec((D, PB), lambda i: (0, i)),
            pl.BlockSpec((1, D), lambda i: (0, 0)),
            pl.BlockSpec((1, 1), lambda i: (0, 0)),
        ],
        out_specs=pl.BlockSpec((PB,), lambda i: (i,)),
        out_shape=jax.ShapeDtypeStruct((VPAD,), jnp.float32),
    )(table_t, w_row, b.reshape(1, 1))


def _sc_body(p_hbm, xt_hbm, out_hbm, p_v, x_v, out_v, sem_p, sem_a, sem_b):
    wid = lax.axis_index("s") * NC + lax.axis_index("c")
    col0 = wid * CPW

    cp_p = pltpu.async_copy(p_hbm, p_v, sem_p)
    sems = (sem_a, sem_b)
    cps = [None, None]
    cps[0] = pltpu.async_copy(xt_hbm.at[:, pl.ds(col0, CH)], x_v.at[0], sem_a)

    for c in range(NCH):
        buf = c & 1
        if c + 1 < NCH:
            nbuf = (c + 1) & 1
            cps[nbuf] = pltpu.async_copy(
                xt_hbm.at[:, pl.ds(col0 + (c + 1) * CH, CH)], x_v.at[nbuf],
                sems[nbuf])
        cps[buf].wait()
        if c == 0:
            cp_p.wait()

        def g_body(g, carry, _buf=buf, _c=c):
            def s_body(s, acc):
                idx = x_v[_buf, s, pl.ds(g * 16, 16)]
                return acc + plsc.load_gather(p_v, [idx])

            acc = lax.fori_loop(0, S, s_body, jnp.zeros((16,), jnp.float32),
                                unroll=10)
            out_v[pl.ds(_c * CH + g * 16, 16)] = acc
            return carry

        lax.fori_loop(0, NG, g_body, 0)

    pltpu.sync_copy(out_v, out_hbm.at[pl.ds(col0, CPW)])


_sc_gather_sum = functools.partial(
    pl.kernel,
    mesh=plsc.VectorSubcoreMesh(core_axis_name="c", subcore_axis_name="s"),
    out_type=jax.ShapeDtypeStruct((B,), jnp.float32),
    compiler_params=pltpu.CompilerParams(needs_layout_passes=False),
    scratch_types=[
        pltpu.VMEM((VPAD,), jnp.float32),
        pltpu.VMEM((2, S, CH), jnp.int32),
        pltpu.VMEM((CPW,), jnp.float32),
        pltpu.SemaphoreType.DMA,
        pltpu.SemaphoreType.DMA,
        pltpu.SemaphoreType.DMA,
    ],
)(_sc_body)


def kernel(x, emb_table, W, b):
    p = _project_table(emb_table.T, W, b)
    out = _sc_gather_sum(p, x.T)
    return out.reshape(B, 1, 1, 1)
